# Initial kernel scaffold; baseline (speedup 1.0000x reference)
import jax
import jax.numpy as jnp
from jax.experimental import pallas as pl


def kernel(inputs, adj_mx, forward_index, sdist, W_gate, b_gate, W_cand, b_cand, W_in0, W_h0, b0, W_in1, W_h1, b1, fc_W, fc_b, bn_gamma, bn_beta, fco_W, fco_b, fcc_W, fcc_b):
    raise NotImplementedError("write your pallas kernel here")


# speedup vs baseline: 3.5629x; 3.5629x over previous
import functools

import jax
import jax.numpy as jnp
from jax.experimental import pallas as pl
from jax.experimental.pallas import tpu as pltpu

N = 2048
U = 128
ID = 128
B = 4
S = 100
SP = 128
ND = 256
ED = 128
NM = 5
W512 = B * ID


def _dot(a, b, trans_lhs=False):
    dims = (((0,) if trans_lhs else (1,), (0,)), ((), ()))
    return jax.lax.dot_general(a, b, dims,
                               preferred_element_type=jnp.float32)


def _sums_kernel(a_ref, d1_ref, d2_ref):
    i = pl.program_id(0)
    a = a_ref[...]
    d1_ref[...] = jnp.sum(a, axis=1, keepdims=True)
    part = jnp.sum(a, axis=0, keepdims=True)

    @pl.when(i == 0)
    def _():
        d2_ref[...] = part

    @pl.when(i != 0)
    def _():
        d2_ref[...] += part


def _dinv(d):
    return jnp.where(d > 0.0, 1.0 / d, 0.0)


def _diff_mm_kernel(trans_lhs, nk, cheb, a_ref, d_ref, x_ref, x0_ref,
                    out_ref):
    k = pl.program_id(0)
    sup = _dinv(d_ref[...]) * a_ref[...]
    part = _dot(sup, x_ref[...], trans_lhs)

    @pl.when(k == 0)
    def _():
        out_ref[...] = part

    @pl.when(k != 0)
    def _():
        out_ref[...] += part

    if cheb:
        @pl.when(k == nk - 1)
        def _():
            out_ref[...] = 2.0 * out_ref[...] - x0_ref[...]


def _fc_gru_kernel(x0_ref, x1a_ref, x2a_ref, x1b_ref, x2b_ref, w_ref, b_ref,
                   out_ref):
    xs = (x0_ref, x1a_ref, x2a_ref, x1b_ref, x2b_ref)
    for b in range(B):
        sl = slice(b * ID, (b + 1) * ID)
        acc = _dot(xs[0][:, sl], w_ref[0])
        for m in range(1, NM):
            acc = acc + _dot(xs[m][:, sl], w_ref[m])
        acc = acc + b_ref[...]
        u = jax.nn.sigmoid(acc[:, :U])
        c = jnp.tanh(acc[:, U:])
        out_ref[b] = (1.0 - u) * c


def _row_sum(e):
    acc = e[:, 0:128]
    for j in range(1, e.shape[1] // 128):
        acc = acc + e[:, 128 * j:128 * (j + 1)]
    w = 128
    while w > 1:
        acc = acc[:, :w // 2] + acc[:, w // 2:]
        w //= 2
    return acc


def _attention_kernel(prev_ref, curr_ref, attn_ref, ctx_ref):
    scale = 1.0 / jnp.sqrt(jnp.float32(S))
    scores = scale * jax.lax.dot_general(
        prev_ref[...], curr_ref[...], (((1,), (1,)), ((), ())),
        preferred_element_type=jnp.float32)
    mx = jnp.max(scores, axis=1, keepdims=True)
    e = jnp.exp(scores - mx)
    attn = e / _row_sum(e)
    attn_ref[...] = attn
    ctx_ref[...] = _dot(attn, curr_ref[...])


def _head_kernel(ctx_ref, wi0_ref, b0_ref, wi1_ref, wh1_ref, b1_ref,
                 fcw_ref, fcb_ref, g_ref, beta_ref, fow_ref, fob_ref,
                 fcc_ref, fccb_ref, ne_ref, xf_ref):
    ctx = ctx_ref[...]
    ne1 = jnp.tanh(_dot(ctx, wi0_ref[...]) + b0_ref[...])
    ne = jnp.tanh(_dot(ctx, wi1_ref[...]) + _dot(ne1, wh1_ref[...])
                  + b1_ref[...])
    ne_ref[...] = ne
    x = jnp.maximum(_dot(ne, fcw_ref[...]) + fcb_ref[...], 0.0)
    mean = jnp.mean(x, axis=0, keepdims=True)
    var = jnp.mean((x - mean) ** 2, axis=0, keepdims=True)
    x = (x - mean) / jnp.sqrt(var + 1e-5) * g_ref[...] + beta_ref[...]
    x = jnp.maximum(_dot(x, fow_ref[...]) + fob_ref[...], 0.0)
    xf_ref[...] = _dot(x, fcc_ref[...]) + fccb_ref[...]


def _outer_kernel(xr_ref, xa_ref, adj_ref):
    adj_ref[...] = jax.lax.dot_general(
        xr_ref[:, 0:1], xa_ref[:, 1:2], (((1,), (1,)), ((), ())),
        preferred_element_type=jnp.float32)


def kernel(inputs, adj_mx, forward_index, sdist, W_gate, b_gate, W_cand,
           b_cand, W_in0, W_h0, b0, W_in1, W_h1, b1, fc_W, fc_b, bn_gamma,
           bn_beta, fco_W, fco_b, fcc_W, fcc_b):
    f32 = jnp.float32

    x0 = inputs.reshape(B, N, ID).transpose(1, 0, 2).reshape(N, W512)
    wg = W_gate.reshape(2 * ID, NM, 2 * U)[:ID, :, U:]
    wc = W_cand.reshape(2 * ID, NM, U)[:ID]
    w = jnp.concatenate([wg, wc], axis=2).transpose(1, 0, 2)
    bias = jnp.concatenate([b_gate[U:], b_cand]).reshape(1, 2 * U)

    BR = 256
    d1, d2 = pl.pallas_call(
        _sums_kernel,
        grid=(N // BR,),
        in_specs=[pl.BlockSpec((BR, N), lambda i: (i, 0))],
        out_specs=[pl.BlockSpec((BR, 1), lambda i: (i, 0)),
                   pl.BlockSpec((1, N), lambda i: (0, 0))],
        out_shape=[jax.ShapeDtypeStruct((N, 1), f32),
                   jax.ShapeDtypeStruct((1, N), f32)],
    )(adj_mx)

    BK = 512
    NK = N // BK

    def diff_mm(x, trans_lhs, cheb):
        body = functools.partial(_diff_mm_kernel, trans_lhs, NK, cheb)
        if trans_lhs:
            a_spec = pl.BlockSpec((BK, N), lambda k: (k, 0))
            d_spec = pl.BlockSpec((BK, 1), lambda k: (k, 0))
            d_arr = d1
        else:
            a_spec = pl.BlockSpec((N, BK), lambda k: (0, k))
            d_spec = pl.BlockSpec((1, BK), lambda k: (0, k))
            d_arr = d2
        return pl.pallas_call(
            body,
            grid=(NK,),
            in_specs=[
                a_spec,
                d_spec,
                pl.BlockSpec((BK, W512), lambda k: (k, 0)),
                pl.BlockSpec((N, W512), lambda k: (0, 0)),
            ],
            out_specs=pl.BlockSpec((N, W512), lambda k: (0, 0)),
            out_shape=jax.ShapeDtypeStruct((N, W512), f32),
            compiler_params=pltpu.CompilerParams(
                vmem_limit_bytes=56 * 1024 * 1024),
        )(adj_mx, d_arr, x, x0)

    x1a = diff_mm(x0, True, False)
    x2a = diff_mm(x1a, True, True)
    x1b = diff_mm(x0, False, False)
    x2b = diff_mm(x1b, False, True)

    BT = 256
    ns = pl.pallas_call(
        _fc_gru_kernel,
        grid=(N // BT,),
        in_specs=[pl.BlockSpec((BT, W512), lambda i: (i, 0))] * 5 + [
            pl.BlockSpec((NM, ID, 2 * U), lambda i: (0, 0, 0)),
            pl.BlockSpec((1, 2 * U), lambda i: (0, 0)),
        ],
        out_specs=pl.BlockSpec((B, BT, U), lambda i: (0, i, 0)),
        out_shape=jax.ShapeDtypeStruct((B, N, U), f32),
    )(x0, x1a, x2a, x1b, x2b, w, bias)
    output = ns.reshape(B, N * U)
    hidden_states = output[None]

    fi = jnp.asarray(forward_index)
    prev = jax.lax.dynamic_index_in_dim(sdist, fi - 1, axis=1, keepdims=False)
    curr = jax.lax.dynamic_index_in_dim(sdist, fi, axis=1, keepdims=False)
    pad = ((0, 0), (0, SP - S))
    prev_p = jnp.pad(prev, pad)
    curr_p = jnp.pad(curr, pad)

    attn, ctx = pl.pallas_call(
        _attention_kernel,
        grid=(N // BT,),
        in_specs=[
            pl.BlockSpec((BT, SP), lambda i: (i, 0)),
            pl.BlockSpec((N, SP), lambda i: (0, 0)),
        ],
        out_specs=[
            pl.BlockSpec((BT, N), lambda i: (i, 0)),
            pl.BlockSpec((BT, SP), lambda i: (i, 0)),
        ],
        out_shape=[
            jax.ShapeDtypeStruct((N, N), f32),
            jax.ShapeDtypeStruct((N, SP), f32),
        ],
    )(prev_p, curr_p)

    wi0 = jnp.pad(W_in0, ((0, SP - S), (0, 0)))
    wi1 = jnp.pad(W_in1, ((0, SP - S), (0, 0)))
    ne, xf = pl.pallas_call(
        _head_kernel,
        out_shape=[
            jax.ShapeDtypeStruct((N, ND), f32),
            jax.ShapeDtypeStruct((N, 2), f32),
        ],
    )(ctx, wi0, b0.reshape(1, ND), wi1, W_h1, b1.reshape(1, ND),
      fc_W, fc_b.reshape(1, ED), bn_gamma.reshape(1, ED),
      bn_beta.reshape(1, ED), fco_W, fco_b.reshape(1, ED),
      fcc_W, fcc_b.reshape(1, 2))

    adj = pl.pallas_call(
        _outer_kernel,
        grid=(N // BT,),
        in_specs=[
            pl.BlockSpec((BT, 2), lambda i: (i, 0)),
            pl.BlockSpec((N, 2), lambda i: (0, 0)),
        ],
        out_specs=pl.BlockSpec((BT, N), lambda i: (i, 0)),
        out_shape=jax.ShapeDtypeStruct((N, N), f32),
    )(xf, xf)

    return (output, hidden_states, adj, ne, attn)


# speedup vs baseline: 4.6686x; 1.3104x over previous
import jax
import jax.numpy as jnp
from jax.experimental import pallas as pl
from jax.experimental.pallas import tpu as pltpu

N = 2048
U = 128
ID = 128
B = 4
S = 100
SP = 128
ND = 256
ED = 128
NM = 5
W512 = B * ID


def _dot(a, b, trans_lhs=False):
    dims = (((0,) if trans_lhs else (1,), (0,)), ((), ()))
    return jax.lax.dot_general(a, b, dims,
                               preferred_element_type=jnp.float32)


def _dinv(d):
    return jnp.where(d > 0.0, 1.0 / d, 0.0)


BKD = 512
NKD = N // BKD


def _diffusion_gru_kernel(a_ref, x0_ref, w_ref, b_ref, ns_ref,
                          x1a, x1b, x2a, x2b, d1s, d2s):
    p = pl.program_id(0)
    k = pl.program_id(1)
    rows = pl.ds(k * BKD, BKD)

    @pl.when(p == 0)
    def _():
        a = a_ref[...]
        d1s[rows, :] = jnp.sum(a, axis=1, keepdims=True)
        part = jnp.sum(a, axis=0, keepdims=True)

        @pl.when(k == 0)
        def _():
            d2s[...] = part

        @pl.when(k != 0)
        def _():
            d2s[...] += part

    @pl.when(p == 1)
    def _():
        a = a_ref[...]
        sup0 = _dinv(d1s[rows, :]) * a
        part = _dot(sup0, x0_ref[rows, :], trans_lhs=True)

        @pl.when(k == 0)
        def _():
            x1a[...] = part

        @pl.when(k != 0)
        def _():
            x1a[...] += part

        sup1 = a * _dinv(d2s[...])
        x1b[rows, :] = _dot(sup1, x0_ref[...])

    @pl.when(p == 2)
    def _():
        a = a_ref[...]
        sup0 = _dinv(d1s[rows, :]) * a
        part = _dot(sup0, x1a[rows, :], trans_lhs=True)

        @pl.when(k == 0)
        def _():
            x2a[...] = part

        @pl.when(k != 0)
        def _():
            x2a[...] += part

        @pl.when(k == NKD - 1)
        def _():
            x2a[...] = 2.0 * x2a[...] - x0_ref[...]

        sup1 = a * _dinv(d2s[...])
        x2b[rows, :] = 2.0 * _dot(sup1, x1b[...]) - x0_ref[rows, :]

    @pl.when(p == 3)
    def _():
        xs = (x0_ref[rows, :], x1a[rows, :], x2a[rows, :],
              x1b[rows, :], x2b[rows, :])
        for b in range(B):
            sl = slice(b * ID, (b + 1) * ID)
            acc = _dot(xs[0][:, sl], w_ref[0])
            for m in range(1, NM):
                acc = acc + _dot(xs[m][:, sl], w_ref[m])
            acc = acc + b_ref[...]
            u = jax.nn.sigmoid(acc[:, :U])
            c = jnp.tanh(acc[:, U:])
            ns_ref[b] = (1.0 - u) * c


def _row_sum(e):
    acc = e[:, 0:128]
    for j in range(1, e.shape[1] // 128):
        acc = acc + e[:, 128 * j:128 * (j + 1)]
    w = 128
    while w > 1:
        acc = acc[:, :w // 2] + acc[:, w // 2:]
        w //= 2
    return acc


def _attention_kernel(prev_ref, curr_ref, attn_ref, ctx_ref):
    scale = 1.0 / jnp.sqrt(jnp.float32(S))
    scores = scale * jax.lax.dot_general(
        prev_ref[...], curr_ref[...], (((1,), (1,)), ((), ())),
        preferred_element_type=jnp.float32)
    mx = jnp.max(scores, axis=1, keepdims=True)
    e = jnp.exp(scores - mx)
    attn = e / _row_sum(e)
    attn_ref[...] = attn
    ctx_ref[...] = _dot(attn, curr_ref[...])


def _head_kernel(ctx_ref, wi0_ref, b0_ref, wi1_ref, wh1_ref, b1_ref,
                 fcw_ref, fcb_ref, g_ref, beta_ref, fow_ref, fob_ref,
                 fcc_ref, fccb_ref, ne_ref, xf_ref):
    ctx = ctx_ref[...]
    ne1 = jnp.tanh(_dot(ctx, wi0_ref[...]) + b0_ref[...])
    ne = jnp.tanh(_dot(ctx, wi1_ref[...]) + _dot(ne1, wh1_ref[...])
                  + b1_ref[...])
    ne_ref[...] = ne
    x = jnp.maximum(_dot(ne, fcw_ref[...]) + fcb_ref[...], 0.0)
    mean = jnp.mean(x, axis=0, keepdims=True)
    var = jnp.mean((x - mean) ** 2, axis=0, keepdims=True)
    x = (x - mean) / jnp.sqrt(var + 1e-5) * g_ref[...] + beta_ref[...]
    x = jnp.maximum(_dot(x, fow_ref[...]) + fob_ref[...], 0.0)
    xf_ref[...] = _dot(x, fcc_ref[...]) + fccb_ref[...]


def _outer_kernel(xr_ref, xa_ref, adj_ref):
    adj_ref[...] = jax.lax.dot_general(
        xr_ref[:, 0:1], xa_ref[:, 1:2], (((1,), (1,)), ((), ())),
        preferred_element_type=jnp.float32)


def kernel(inputs, adj_mx, forward_index, sdist, W_gate, b_gate, W_cand,
           b_cand, W_in0, W_h0, b0, W_in1, W_h1, b1, fc_W, fc_b, bn_gamma,
           bn_beta, fco_W, fco_b, fcc_W, fcc_b):
    f32 = jnp.float32

    x0 = inputs.reshape(B, N, ID).transpose(1, 0, 2).reshape(N, W512)
    wg = W_gate.reshape(2 * ID, NM, 2 * U)[:ID, :, U:]
    wc = W_cand.reshape(2 * ID, NM, U)[:ID]
    w = jnp.concatenate([wg, wc], axis=2).transpose(1, 0, 2)
    bias = jnp.concatenate([b_gate[U:], b_cand]).reshape(1, 2 * U)

    ns = pl.pallas_call(
        _diffusion_gru_kernel,
        grid=(4, NKD),
        in_specs=[
            pl.BlockSpec((BKD, N),
                         lambda p, k: (jnp.where(p == 3, 0, k), 0)),
            pl.BlockSpec((N, W512), lambda p, k: (0, 0)),
            pl.BlockSpec((NM, ID, 2 * U), lambda p, k: (0, 0, 0)),
            pl.BlockSpec((1, 2 * U), lambda p, k: (0, 0)),
        ],
        out_specs=pl.BlockSpec((B, BKD, U),
                               lambda p, k: (0, jnp.where(p == 3, k, 0), 0)),
        out_shape=jax.ShapeDtypeStruct((B, N, U), f32),
        scratch_shapes=[
            pltpu.VMEM((N, W512), f32),
            pltpu.VMEM((N, W512), f32),
            pltpu.VMEM((N, W512), f32),
            pltpu.VMEM((N, W512), f32),
            pltpu.VMEM((N, 1), f32),
            pltpu.VMEM((1, N), f32),
        ],
        compiler_params=pltpu.CompilerParams(
            vmem_limit_bytes=56 * 1024 * 1024),
    )(adj_mx, x0, w, bias)
    output = ns.reshape(B, N * U)
    hidden_states = output[None]

    fi = jnp.asarray(forward_index)
    prev = jax.lax.dynamic_index_in_dim(sdist, fi - 1, axis=1, keepdims=False)
    curr = jax.lax.dynamic_index_in_dim(sdist, fi, axis=1, keepdims=False)
    pad = ((0, 0), (0, SP - S))
    prev_p = jnp.pad(prev, pad)
    curr_p = jnp.pad(curr, pad)

    BT = 256
    attn, ctx = pl.pallas_call(
        _attention_kernel,
        grid=(N // BT,),
        in_specs=[
            pl.BlockSpec((BT, SP), lambda i: (i, 0)),
            pl.BlockSpec((N, SP), lambda i: (0, 0)),
        ],
        out_specs=[
            pl.BlockSpec((BT, N), lambda i: (i, 0)),
            pl.BlockSpec((BT, SP), lambda i: (i, 0)),
        ],
        out_shape=[
            jax.ShapeDtypeStruct((N, N), f32),
            jax.ShapeDtypeStruct((N, SP), f32),
        ],
    )(prev_p, curr_p)

    wi0 = jnp.pad(W_in0, ((0, SP - S), (0, 0)))
    wi1 = jnp.pad(W_in1, ((0, SP - S), (0, 0)))
    ne, xf = pl.pallas_call(
        _head_kernel,
        out_shape=[
            jax.ShapeDtypeStruct((N, ND), f32),
            jax.ShapeDtypeStruct((N, 2), f32),
        ],
    )(ctx, wi0, b0.reshape(1, ND), wi1, W_h1, b1.reshape(1, ND),
      fc_W, fc_b.reshape(1, ED), bn_gamma.reshape(1, ED),
      bn_beta.reshape(1, ED), fco_W, fco_b.reshape(1, ED),
      fcc_W, fcc_b.reshape(1, 2))

    adj = pl.pallas_call(
        _outer_kernel,
        grid=(N // BT,),
        in_specs=[
            pl.BlockSpec((BT, 2), lambda i: (i, 0)),
            pl.BlockSpec((N, 2), lambda i: (0, 0)),
        ],
        out_specs=pl.BlockSpec((BT, N), lambda i: (i, 0)),
        out_shape=jax.ShapeDtypeStruct((N, N), f32),
    )(xf, xf)

    return (output, hidden_states, adj, ne, attn)


# speedup vs baseline: 4.7907x; 1.0262x over previous
import jax
import jax.numpy as jnp
from jax.experimental import pallas as pl
from jax.experimental.pallas import tpu as pltpu

N = 2048
U = 128
ID = 128
B = 4
S = 100
SP = 128
ND = 256
ED = 128
NM = 5
W512 = B * ID


def _dot(a, b, trans_lhs=False):
    dims = (((0,) if trans_lhs else (1,), (0,)), ((), ()))
    return jax.lax.dot_general(a, b, dims,
                               preferred_element_type=jnp.float32)


def _dinv(d):
    return jnp.where(d > 0.0, 1.0 / d, 0.0)


BKD = 512
NKD = N // BKD


def _diffusion_gru_kernel(a_ref, x0_ref, w_ref, b_ref, ns_ref,
                          x1a, x1b, x2a, x2b, d1s, d2s):
    p = pl.program_id(0)
    k = pl.program_id(1)
    rows = pl.ds(k * BKD, BKD)

    @pl.when(p == 0)
    def _():
        a = a_ref[...]
        d1s[rows, :] = jnp.sum(a, axis=1, keepdims=True)
        part = jnp.sum(a, axis=0, keepdims=True)

        @pl.when(k == 0)
        def _():
            d2s[...] = part

        @pl.when(k != 0)
        def _():
            d2s[...] += part

    @pl.when(p == 1)
    def _():
        a = a_ref[...]
        sup0 = _dinv(d1s[rows, :]) * a
        part = _dot(sup0, x0_ref[rows, :], trans_lhs=True)

        @pl.when(k == 0)
        def _():
            x1a[...] = part

        @pl.when(k != 0)
        def _():
            x1a[...] += part

        sup1 = a * _dinv(d2s[...])
        x1b[rows, :] = _dot(sup1, x0_ref[...])

    @pl.when(p == 2)
    def _():
        a = a_ref[...]
        sup0 = _dinv(d1s[rows, :]) * a
        part = _dot(sup0, x1a[rows, :], trans_lhs=True)

        @pl.when(k == 0)
        def _():
            x2a[...] = part

        @pl.when(k != 0)
        def _():
            x2a[...] += part

        @pl.when(k == NKD - 1)
        def _():
            x2a[...] = 2.0 * x2a[...] - x0_ref[...]

        sup1 = a * _dinv(d2s[...])
        x2b[rows, :] = 2.0 * _dot(sup1, x1b[...]) - x0_ref[rows, :]

    @pl.when(p == 3)
    def _():
        xs = (x0_ref[rows, :], x1a[rows, :], x2a[rows, :],
              x1b[rows, :], x2b[rows, :])
        for b in range(B):
            sl = slice(b * ID, (b + 1) * ID)
            acc = _dot(xs[0][:, sl], w_ref[0])
            for m in range(1, NM):
                acc = acc + _dot(xs[m][:, sl], w_ref[m])
            acc = acc + b_ref[...]
            u = jax.nn.sigmoid(acc[:, :U])
            c = jnp.tanh(acc[:, U:])
            ns_ref[b] = (1.0 - u) * c


def _row_sum(e):
    acc = e[:, 0:128]
    for j in range(1, e.shape[1] // 128):
        acc = acc + e[:, 128 * j:128 * (j + 1)]
    w = 128
    while w > 1:
        acc = acc[:, :w // 2] + acc[:, w // 2:]
        w //= 2
    return acc


BTA = 256
NTA = N // BTA


def _attn_head_kernel(prev_ref, curr_ref, wi0_ref, b0_ref, wi1_ref, wh1_ref,
                      b1_ref, fcw_ref, fcb_ref, g_ref, beta_ref, fow_ref,
                      fob_ref, fcc_ref, fccb_ref, attn_ref, ne_ref, adj_ref,
                      ctx_s, xf_s):
    p = pl.program_id(0)
    k = pl.program_id(1)

    @pl.when(p == 0)
    def _():
        scale = 1.0 / jnp.sqrt(jnp.float32(S))
        scores = scale * jax.lax.dot_general(
            prev_ref[...], curr_ref[...], (((1,), (1,)), ((), ())),
            preferred_element_type=jnp.float32)
        mx = jnp.max(scores, axis=1, keepdims=True)
        e = jnp.exp(scores - mx)
        attn = e / _row_sum(e)
        attn_ref[...] = attn
        ctx_s[pl.ds(k * BTA, BTA), :] = _dot(attn, curr_ref[...])

    @pl.when(jnp.logical_and(p == 1, k == 0))
    def _():
        ctx = ctx_s[...]
        ne1 = jnp.tanh(_dot(ctx, wi0_ref[...]) + b0_ref[...])
        ne = jnp.tanh(_dot(ctx, wi1_ref[...]) + _dot(ne1, wh1_ref[...])
                      + b1_ref[...])
        ne_ref[...] = ne
        x = jnp.maximum(_dot(ne, fcw_ref[...]) + fcb_ref[...], 0.0)
        mean = jnp.mean(x, axis=0, keepdims=True)
        var = jnp.mean((x - mean) ** 2, axis=0, keepdims=True)
        x = (x - mean) / jnp.sqrt(var + 1e-5) * g_ref[...] + beta_ref[...]
        x = jnp.maximum(_dot(x, fow_ref[...]) + fob_ref[...], 0.0)
        xf_s[...] = _dot(x, fcc_ref[...]) + fccb_ref[...]

    @pl.when(p == 1)
    def _():
        xr = xf_s[pl.ds(k * BTA, BTA), :]
        adj_ref[...] = jax.lax.dot_general(
            xr[:, 0:1], xf_s[:, 1:2], (((1,), (1,)), ((), ())),
            preferred_element_type=jnp.float32)


def kernel(inputs, adj_mx, forward_index, sdist, W_gate, b_gate, W_cand,
           b_cand, W_in0, W_h0, b0, W_in1, W_h1, b1, fc_W, fc_b, bn_gamma,
           bn_beta, fco_W, fco_b, fcc_W, fcc_b):
    f32 = jnp.float32

    x0 = inputs.reshape(B, N, ID).transpose(1, 0, 2).reshape(N, W512)
    wg = W_gate.reshape(2 * ID, NM, 2 * U)[:ID, :, U:]
    wc = W_cand.reshape(2 * ID, NM, U)[:ID]
    w = jnp.concatenate([wg, wc], axis=2).transpose(1, 0, 2)
    bias = jnp.concatenate([b_gate[U:], b_cand]).reshape(1, 2 * U)

    ns = pl.pallas_call(
        _diffusion_gru_kernel,
        grid=(4, NKD),
        in_specs=[
            pl.BlockSpec((BKD, N),
                         lambda p, k: (jnp.where(p == 3, 0, k), 0)),
            pl.BlockSpec((N, W512), lambda p, k: (0, 0)),
            pl.BlockSpec((NM, ID, 2 * U), lambda p, k: (0, 0, 0)),
            pl.BlockSpec((1, 2 * U), lambda p, k: (0, 0)),
        ],
        out_specs=pl.BlockSpec((B, BKD, U),
                               lambda p, k: (0, jnp.where(p == 3, k, 0), 0)),
        out_shape=jax.ShapeDtypeStruct((B, N, U), f32),
        scratch_shapes=[
            pltpu.VMEM((N, W512), f32),
            pltpu.VMEM((N, W512), f32),
            pltpu.VMEM((N, W512), f32),
            pltpu.VMEM((N, W512), f32),
            pltpu.VMEM((N, 1), f32),
            pltpu.VMEM((1, N), f32),
        ],
        compiler_params=pltpu.CompilerParams(
            vmem_limit_bytes=56 * 1024 * 1024),
    )(adj_mx, x0, w, bias)
    output = ns.reshape(B, N * U)
    hidden_states = output[None]

    fi = jnp.asarray(forward_index)
    prev = jax.lax.dynamic_index_in_dim(sdist, fi - 1, axis=1, keepdims=False)
    curr = jax.lax.dynamic_index_in_dim(sdist, fi, axis=1, keepdims=False)
    pad = ((0, 0), (0, SP - S))
    prev_p = jnp.pad(prev, pad)
    curr_p = jnp.pad(curr, pad)

    wi0 = jnp.pad(W_in0, ((0, SP - S), (0, 0)))
    wi1 = jnp.pad(W_in1, ((0, SP - S), (0, 0)))
    const2 = lambda p, k: (0, 0)
    attn, ne, adj = pl.pallas_call(
        _attn_head_kernel,
        grid=(2, NTA),
        in_specs=[
            pl.BlockSpec((BTA, SP),
                         lambda p, k: (jnp.where(p == 0, k, 0), 0)),
            pl.BlockSpec((N, SP), const2),
            pl.BlockSpec((SP, ND), const2),
            pl.BlockSpec((1, ND), const2),
            pl.BlockSpec((SP, ND), const2),
            pl.BlockSpec((ND, ND), const2),
            pl.BlockSpec((1, ND), const2),
            pl.BlockSpec((ND, ED), const2),
            pl.BlockSpec((1, ED), const2),
            pl.BlockSpec((1, ED), const2),
            pl.BlockSpec((1, ED), const2),
            pl.BlockSpec((ED, ED), const2),
            pl.BlockSpec((1, ED), const2),
            pl.BlockSpec((ED, 2), const2),
            pl.BlockSpec((1, 2), const2),
        ],
        out_specs=[
            pl.BlockSpec((BTA, N),
                         lambda p, k: (jnp.where(p == 0, k, NTA - 1), 0)),
            pl.BlockSpec((N, ND), const2),
            pl.BlockSpec((BTA, N),
                         lambda p, k: (jnp.where(p == 1, k, 0), 0)),
        ],
        out_shape=[
            jax.ShapeDtypeStruct((N, N), f32),
            jax.ShapeDtypeStruct((N, ND), f32),
            jax.ShapeDtypeStruct((N, N), f32),
        ],
        scratch_shapes=[
            pltpu.VMEM((N, SP), f32),
            pltpu.VMEM((N, 2), f32),
        ],
    )(prev_p, curr_p, wi0, b0.reshape(1, ND), wi1, W_h1, b1.reshape(1, ND),
      fc_W, fc_b.reshape(1, ED), bn_gamma.reshape(1, ED),
      bn_beta.reshape(1, ED), fco_W, fco_b.reshape(1, ED),
      fcc_W, fcc_b.reshape(1, 2))

    return (output, hidden_states, adj, ne, attn)


# speedup vs baseline: 4.9330x; 1.0297x over previous
import jax
import jax.numpy as jnp
from jax.experimental import pallas as pl
from jax.experimental.pallas import tpu as pltpu

N = 2048
U = 128
ID = 128
B = 4
S = 100
SP = 128
ND = 256
ED = 128
NM = 5
W512 = B * ID


def _dot(a, b, trans_lhs=False):
    dims = (((0,) if trans_lhs else (1,), (0,)), ((), ()))
    return jax.lax.dot_general(a, b, dims,
                               preferred_element_type=jnp.float32)


def _dinv(d):
    return jnp.where(d > 0.0, 1.0 / d, 0.0)


BKD = 1024
NKD = N // BKD


def _diffusion_gru_kernel(a_ref, x0_ref, w_ref, b_ref, ns_ref,
                          x1a, x1b, x2a, x2b, d1s, d2s):
    p = pl.program_id(0)
    k = pl.program_id(1)
    rows = pl.ds(k * BKD, BKD)

    @pl.when(p == 0)
    def _():
        a = a_ref[...]
        d1s[rows, :] = jnp.sum(a, axis=1, keepdims=True)
        part = jnp.sum(a, axis=0, keepdims=True)

        @pl.when(k == 0)
        def _():
            d2s[...] = part

        @pl.when(k != 0)
        def _():
            d2s[...] += part

    @pl.when(p == 1)
    def _():
        a = a_ref[...]
        sup0 = _dinv(d1s[rows, :]) * a
        part = _dot(sup0, x0_ref[rows, :], trans_lhs=True)

        @pl.when(k == 0)
        def _():
            x1a[...] = part

        @pl.when(k != 0)
        def _():
            x1a[...] += part

        sup1 = a * _dinv(d2s[...])
        x1b[rows, :] = _dot(sup1, x0_ref[...])

    @pl.when(p == 2)
    def _():
        a = a_ref[...]
        sup0 = _dinv(d1s[rows, :]) * a
        part = _dot(sup0, x1a[rows, :], trans_lhs=True)

        @pl.when(k == 0)
        def _():
            x2a[...] = part

        @pl.when(k != 0)
        def _():
            x2a[...] += part

        @pl.when(k == NKD - 1)
        def _():
            x2a[...] = 2.0 * x2a[...] - x0_ref[...]

        sup1 = a * _dinv(d2s[...])
        x2b[rows, :] = 2.0 * _dot(sup1, x1b[...]) - x0_ref[rows, :]

    @pl.when(p == 3)
    def _():
        xs = (x0_ref[rows, :], x1a[rows, :], x2a[rows, :],
              x1b[rows, :], x2b[rows, :])
        for b in range(B):
            sl = slice(b * ID, (b + 1) * ID)
            acc = _dot(xs[0][:, sl], w_ref[0])
            for m in range(1, NM):
                acc = acc + _dot(xs[m][:, sl], w_ref[m])
            acc = acc + b_ref[...]
            u = jax.nn.sigmoid(acc[:, :U])
            c = jnp.tanh(acc[:, U:])
            ns_ref[b] = (1.0 - u) * c


def _row_sum(e):
    acc = e[:, 0:128]
    for j in range(1, e.shape[1] // 128):
        acc = acc + e[:, 128 * j:128 * (j + 1)]
    w = 128
    while w > 1:
        acc = acc[:, :w // 2] + acc[:, w // 2:]
        w //= 2
    return acc


BTA = 256
NTA = N // BTA


def _attn_head_kernel(prev_ref, curr_ref, wi0_ref, b0_ref, wi1_ref, wh1_ref,
                      b1_ref, fcw_ref, fcb_ref, g_ref, beta_ref, fow_ref,
                      fob_ref, fcc_ref, fccb_ref, attn_ref, ne_ref, adj_ref,
                      ctx_s, xf_s):
    p = pl.program_id(0)
    k = pl.program_id(1)

    @pl.when(p == 0)
    def _():
        scale = 1.0 / jnp.sqrt(jnp.float32(S))
        scores = scale * jax.lax.dot_general(
            prev_ref[...], curr_ref[...], (((1,), (1,)), ((), ())),
            preferred_element_type=jnp.float32)
        mx = jnp.max(scores, axis=1, keepdims=True)
        e = jnp.exp(scores - mx)
        attn = e / _row_sum(e)
        attn_ref[...] = attn
        ctx_s[pl.ds(k * BTA, BTA), :] = _dot(attn, curr_ref[...])

    @pl.when(jnp.logical_and(p == 1, k == 0))
    def _():
        ctx = ctx_s[...]
        ne1 = jnp.tanh(_dot(ctx, wi0_ref[...]) + b0_ref[...])
        ne = jnp.tanh(_dot(ctx, wi1_ref[...]) + _dot(ne1, wh1_ref[...])
                      + b1_ref[...])
        ne_ref[...] = ne
        x = jnp.maximum(_dot(ne, fcw_ref[...]) + fcb_ref[...], 0.0)
        mean = jnp.mean(x, axis=0, keepdims=True)
        var = jnp.mean((x - mean) ** 2, axis=0, keepdims=True)
        x = (x - mean) / jnp.sqrt(var + 1e-5) * g_ref[...] + beta_ref[...]
        x = jnp.maximum(_dot(x, fow_ref[...]) + fob_ref[...], 0.0)
        xf_s[...] = _dot(x, fcc_ref[...]) + fccb_ref[...]

    @pl.when(p == 1)
    def _():
        xr = xf_s[pl.ds(k * BTA, BTA), :]
        adj_ref[...] = jax.lax.dot_general(
            xr[:, 0:1], xf_s[:, 1:2], (((1,), (1,)), ((), ())),
            preferred_element_type=jnp.float32)


def kernel(inputs, adj_mx, forward_index, sdist, W_gate, b_gate, W_cand,
           b_cand, W_in0, W_h0, b0, W_in1, W_h1, b1, fc_W, fc_b, bn_gamma,
           bn_beta, fco_W, fco_b, fcc_W, fcc_b):
    f32 = jnp.float32

    x0 = inputs.reshape(B, N, ID).transpose(1, 0, 2).reshape(N, W512)
    wg = W_gate.reshape(2 * ID, NM, 2 * U)[:ID, :, U:]
    wc = W_cand.reshape(2 * ID, NM, U)[:ID]
    w = jnp.concatenate([wg, wc], axis=2).transpose(1, 0, 2)
    bias = jnp.concatenate([b_gate[U:], b_cand]).reshape(1, 2 * U)

    ns = pl.pallas_call(
        _diffusion_gru_kernel,
        grid=(4, NKD),
        in_specs=[
            pl.BlockSpec((BKD, N),
                         lambda p, k: (jnp.where(p == 3, 0, k), 0)),
            pl.BlockSpec((N, W512), lambda p, k: (0, 0)),
            pl.BlockSpec((NM, ID, 2 * U), lambda p, k: (0, 0, 0)),
            pl.BlockSpec((1, 2 * U), lambda p, k: (0, 0)),
        ],
        out_specs=pl.BlockSpec((B, BKD, U),
                               lambda p, k: (0, jnp.where(p == 3, k, 0), 0)),
        out_shape=jax.ShapeDtypeStruct((B, N, U), f32),
        scratch_shapes=[
            pltpu.VMEM((N, W512), f32),
            pltpu.VMEM((N, W512), f32),
            pltpu.VMEM((N, W512), f32),
            pltpu.VMEM((N, W512), f32),
            pltpu.VMEM((N, 1), f32),
            pltpu.VMEM((1, N), f32),
        ],
        compiler_params=pltpu.CompilerParams(
            vmem_limit_bytes=56 * 1024 * 1024),
    )(adj_mx, x0, w, bias)
    output = ns.reshape(B, N * U)
    hidden_states = output[None]

    fi = jnp.asarray(forward_index)
    prev = jax.lax.dynamic_index_in_dim(sdist, fi - 1, axis=1, keepdims=False)
    curr = jax.lax.dynamic_index_in_dim(sdist, fi, axis=1, keepdims=False)
    pad = ((0, 0), (0, SP - S))
    prev_p = jnp.pad(prev, pad)
    curr_p = jnp.pad(curr, pad)

    wi0 = jnp.pad(W_in0, ((0, SP - S), (0, 0)))
    wi1 = jnp.pad(W_in1, ((0, SP - S), (0, 0)))
    const2 = lambda p, k: (0, 0)
    attn, ne, adj = pl.pallas_call(
        _attn_head_kernel,
        grid=(2, NTA),
        in_specs=[
            pl.BlockSpec((BTA, SP),
                         lambda p, k: (jnp.where(p == 0, k, 0), 0)),
            pl.BlockSpec((N, SP), const2),
            pl.BlockSpec((SP, ND), const2),
            pl.BlockSpec((1, ND), const2),
            pl.BlockSpec((SP, ND), const2),
            pl.BlockSpec((ND, ND), const2),
            pl.BlockSpec((1, ND), const2),
            pl.BlockSpec((ND, ED), const2),
            pl.BlockSpec((1, ED), const2),
            pl.BlockSpec((1, ED), const2),
            pl.BlockSpec((1, ED), const2),
            pl.BlockSpec((ED, ED), const2),
            pl.BlockSpec((1, ED), const2),
            pl.BlockSpec((ED, 2), const2),
            pl.BlockSpec((1, 2), const2),
        ],
        out_specs=[
            pl.BlockSpec((BTA, N),
                         lambda p, k: (jnp.where(p == 0, k, NTA - 1), 0)),
            pl.BlockSpec((N, ND), const2),
            pl.BlockSpec((BTA, N),
                         lambda p, k: (jnp.where(p == 1, k, 0), 0)),
        ],
        out_shape=[
            jax.ShapeDtypeStruct((N, N), f32),
            jax.ShapeDtypeStruct((N, ND), f32),
            jax.ShapeDtypeStruct((N, N), f32),
        ],
        scratch_shapes=[
            pltpu.VMEM((N, SP), f32),
            pltpu.VMEM((N, 2), f32),
        ],
    )(prev_p, curr_p, wi0, b0.reshape(1, ND), wi1, W_h1, b1.reshape(1, ND),
      fc_W, fc_b.reshape(1, ED), bn_gamma.reshape(1, ED),
      bn_beta.reshape(1, ED), fco_W, fco_b.reshape(1, ED),
      fcc_W, fcc_b.reshape(1, 2))

    return (output, hidden_states, adj, ne, attn)
